# baseline (speedup 1.0000x reference)
import jax
import jax.numpy as jnp
from jax.experimental import pallas as pl
from jax.experimental.pallas import tpu as pltpu

DIM = 128
QUEUE_SIZE = 65536
BATCH_COLS = 4096


def _copy_body(lk_ref, q_ref, out_ref, sem_q, sem_k):
    tail = pltpu.make_async_copy(
        q_ref.at[:, pl.ds(BATCH_COLS, QUEUE_SIZE - BATCH_COLS)],
        out_ref.at[:, pl.ds(BATCH_COLS, QUEUE_SIZE - BATCH_COLS)],
        sem_q,
    )
    head = pltpu.make_async_copy(
        lk_ref,
        out_ref.at[:, pl.ds(0, BATCH_COLS)],
        sem_k,
    )
    tail.start()
    head.start()
    tail.wait()
    head.wait()


def kernel(last_k, moco_queue):
    return pl.pallas_call(
        _copy_body,
        in_specs=[
            pl.BlockSpec(memory_space=pl.ANY),
            pl.BlockSpec(memory_space=pl.ANY),
        ],
        out_specs=pl.BlockSpec(memory_space=pl.ANY),
        out_shape=jax.ShapeDtypeStruct((DIM, QUEUE_SIZE), jnp.float32),
        scratch_shapes=[pltpu.SemaphoreType.DMA, pltpu.SemaphoreType.DMA],
    )(last_k, moco_queue)


# speedup vs baseline: 40.4842x; 40.4842x over previous
import jax
import jax.numpy as jnp
from jax.experimental import pallas as pl
from jax.experimental.pallas import tpu as pltpu

DIM = 128
QUEUE_SIZE = 65536
BATCH_COLS = 4096

_W = 4096
_GRID = QUEUE_SIZE // _W


def _copy_body(lk_ref, q_ref, out_ref):
    j = pl.program_id(0)

    @pl.when(j == 0)
    def _():
        out_ref[...] = lk_ref[...]

    @pl.when(j > 0)
    def _():
        out_ref[...] = q_ref[...]


def kernel(last_k, moco_queue):
    return pl.pallas_call(
        _copy_body,
        grid=(_GRID,),
        in_specs=[
            pl.BlockSpec((DIM, BATCH_COLS), lambda j: (0, 0)),
            pl.BlockSpec((DIM, _W), lambda j: (0, jnp.maximum(j, 1))),
        ],
        out_specs=pl.BlockSpec((DIM, _W), lambda j: (0, j)),
        out_shape=jax.ShapeDtypeStruct((DIM, QUEUE_SIZE), jnp.float32),
    )(last_k, moco_queue)


# speedup vs baseline: 44.1704x; 1.0911x over previous
import jax
import jax.numpy as jnp
from jax.experimental import pallas as pl
from jax.experimental.pallas import tpu as pltpu

DIM = 128
QUEUE_SIZE = 65536
BATCH_COLS = 4096

_W = 8192
_GRID = QUEUE_SIZE // _W


def _copy_body(lk_ref, q_ref, out_ref):
    j = pl.program_id(0)

    @pl.when(j == 0)
    def _():
        out_ref[:, : BATCH_COLS] = lk_ref[...]
        out_ref[:, BATCH_COLS:] = q_ref[:, BATCH_COLS:]

    @pl.when(j > 0)
    def _():
        out_ref[...] = q_ref[...]


def kernel(last_k, moco_queue):
    return pl.pallas_call(
        _copy_body,
        grid=(_GRID,),
        in_specs=[
            pl.BlockSpec((DIM, BATCH_COLS), lambda j: (0, 0)),
            pl.BlockSpec((DIM, _W), lambda j: (0, j)),
        ],
        out_specs=pl.BlockSpec((DIM, _W), lambda j: (0, j)),
        out_shape=jax.ShapeDtypeStruct((DIM, QUEUE_SIZE), jnp.float32),
    )(last_k, moco_queue)


# speedup vs baseline: 47.0978x; 1.0663x over previous
import jax
import jax.numpy as jnp
from jax.experimental import pallas as pl
from jax.experimental.pallas import tpu as pltpu

DIM = 128
QUEUE_SIZE = 65536
BATCH_COLS = 4096

_W = 16384
_GRID = QUEUE_SIZE // _W


def _copy_body(lk_ref, q_ref, out_ref):
    j = pl.program_id(0)

    @pl.when(j == 0)
    def _():
        out_ref[:, : BATCH_COLS] = lk_ref[...]
        out_ref[:, BATCH_COLS:] = q_ref[:, BATCH_COLS:]

    @pl.when(j > 0)
    def _():
        out_ref[...] = q_ref[...]


def kernel(last_k, moco_queue):
    return pl.pallas_call(
        _copy_body,
        grid=(_GRID,),
        in_specs=[
            pl.BlockSpec((DIM, BATCH_COLS), lambda j: (0, 0)),
            pl.BlockSpec((DIM, _W), lambda j: (0, j)),
        ],
        out_specs=pl.BlockSpec((DIM, _W), lambda j: (0, j)),
        out_shape=jax.ShapeDtypeStruct((DIM, QUEUE_SIZE), jnp.float32),
    )(last_k, moco_queue)
